# baseline (speedup 1.0000x reference)
import functools

import numpy as np

import jax
import jax.numpy as jnp
from jax import lax
from jax.experimental import pallas as pl
from jax.experimental.pallas import tpu as pltpu
from jax.experimental.pallas import tpu_sc as plsc

VOCAB = 1000000
EMB = 64
OUT = 6
PAD = 8
B, L = 4096, 200
N_TOK = B * L

CBLK = 7936
NST = 126
MAIN = NST * CBLK
VPR = 128 // PAD
GRP = CBLK // VPR
T3_ROWS = VOCAB // VPR


def _mlp_body(tt_hbm, mc_ref, bias_ref, out_ref, buf, sem):
    i = pl.program_id(0)

    @pl.when(i == 0)
    def _():
        pltpu.make_async_copy(tt_hbm.at[:, pl.ds(0, CBLK)], buf.at[0],
                              sem.at[0]).start()
        pltpu.make_async_copy(tt_hbm.at[:, pl.ds(CBLK, CBLK)], buf.at[1],
                              sem.at[1]).start()

    @pl.when(i + 2 <= NST - 1)
    def _():
        ns = lax.rem(i + 2, 3)
        pltpu.make_async_copy(tt_hbm.at[:, pl.ds((i + 2) * CBLK, CBLK)],
                              buf.at[ns], sem.at[ns]).start()

    slot = lax.rem(i, 3)
    pltpu.make_async_copy(tt_hbm.at[:, pl.ds(i * CBLK, CBLK)], buf.at[slot],
                          sem.at[slot]).wait()
    mc = mc_ref[...]
    bias = bias_ref[...]
    for m in range(VPR):
        tbm = buf[slot, :, m * GRP:(m + 1) * GRP]
        o = lax.dot_general(tbm, mc, (((0,), (1,)), ((), ())),
                            preferred_element_type=jnp.float32)
        o = o + bias
        out_ref[:, m * PAD:(m + 1) * PAD] = jnp.where(o > 0.0, o,
                                                      jnp.exp(o) - 1.0)


def _transform_table(tt, Mc, bias):
    return pl.pallas_call(
        _mlp_body,
        grid=(NST,),
        in_specs=[
            pl.BlockSpec(memory_space=pl.ANY),
            pl.BlockSpec((PAD, EMB), lambda i: (0, 0)),
            pl.BlockSpec((1, PAD), lambda i: (0, 0)),
        ],
        out_specs=pl.BlockSpec((GRP, 128), lambda i: (i, 0)),
        out_shape=jax.ShapeDtypeStruct((T3_ROWS, 128), jnp.float32),
        scratch_shapes=[
            pltpu.VMEM((3, EMB, CBLK), jnp.float32),
            pltpu.SemaphoreType.DMA((3,)),
        ],
        compiler_params=pltpu.CompilerParams(
            dimension_semantics=("arbitrary",),
        ),
    )(tt, Mc, bias)


def _tail_body(main_ref, tail_ref, out_ref):
    out_ref[...] = tail_ref[...]


def _patch_tail(t3main, tail16):
    return pl.pallas_call(
        _tail_body,
        grid=(1,),
        in_specs=[
            pl.BlockSpec(memory_space=pl.ANY),
            pl.BlockSpec((8, 128), lambda i: (0, 0)),
        ],
        out_specs=pl.BlockSpec((8, 128), lambda i: (T3_ROWS // 8 - 1, 0)),
        out_shape=jax.ShapeDtypeStruct((T3_ROWS, 128), jnp.float32),
        input_output_aliases={0: 0},
    )(t3main, tail16)


NC, NS = 2, 16
NW = NC * NS
PER_W = N_TOK // NW
CH = 3200
N_CH = PER_W // CH


def _gather_body(table_hbm, idx_hbm, out_hbm, idx_v, rows_v, sem):
    wid = lax.axis_index("s") * NC + lax.axis_index("c")
    base = wid * PER_W
    for j in range(N_CH):
        off = base + j * CH
        pltpu.sync_copy(idx_hbm.at[pl.ds(off, CH)], idx_v)
        pltpu.async_copy(table_hbm.at[idx_v], rows_v, sem).wait()
        pltpu.sync_copy(rows_v, out_hbm.at[pl.ds(off, CH)])


@functools.cache
def _make_gather():
    return pl.kernel(
        _gather_body,
        mesh=plsc.VectorSubcoreMesh(core_axis_name="c", subcore_axis_name="s"),
        out_type=jax.ShapeDtypeStruct((N_TOK, PAD), jnp.float32),
        scratch_types=[
            pltpu.VMEM((CH,), jnp.int32),
            pltpu.VMEM((CH, PAD), jnp.float32),
            pltpu.SemaphoreType.DMA,
        ],
        compiler_params=pltpu.CompilerParams(use_tc_tiling_on_sc=False),
    )


def _tail_vlist():
    q = np.arange(T3_ROWS - 8, T3_ROWS)[:, None]
    m = np.arange(VPR)[None, :]
    in_main = q < MAIN // VPR
    qq = q - (NST - 1) * GRP
    v_main = (NST - 1) * CBLK + m * GRP + qq
    v_tail = MAIN + (q - MAIN // VPR) * VPR + m
    return np.where(in_main, v_main, v_tail).astype(np.int32).reshape(-1)


_TAIL_VLIST = _tail_vlist()


def kernel(src, table, W1, b1, W2, b2):
    W2p = jnp.zeros((PAD, EMB), jnp.float32).at[:OUT].set(W2)
    b2p = jnp.zeros((PAD,), jnp.float32).at[:OUT].set(b2)
    Mc = jnp.dot(W2p, W1, precision=lax.Precision.HIGHEST)
    bias16 = W2p @ b1 + b2p
    bias = bias16.reshape(1, PAD)

    t3main = _transform_table(table.T, Mc, bias)

    trows = table[_TAIL_VLIST]
    to = jnp.dot(trows, Mc.T, precision=lax.Precision.HIGHEST) + bias16
    tail16 = jnp.where(to > 0.0, to, jnp.expm1(to)).reshape(8, 128)
    t3 = _patch_tail(t3main, tail16)

    v = src.reshape(N_TOK)
    r = v % CBLK
    idx2 = jnp.where(v < MAIN,
                     ((v // CBLK) * GRP + r % GRP) * VPR + r // GRP,
                     v).astype(jnp.int32)

    rows = _make_gather()(t3.reshape(VOCAB, PAD), idx2)
    return rows[:, :OUT].reshape(B, L, OUT)


# speedup vs baseline: 1.3186x; 1.3186x over previous
import functools

import jax
import jax.numpy as jnp
from jax import lax
from jax.experimental import pallas as pl
from jax.experimental.pallas import tpu as pltpu
from jax.experimental.pallas import tpu_sc as plsc

VOCAB = 1000000
EMB = 64
OUT = 6
PAD = 16
B, L = 4096, 200
N_TOK = B * L

CBLK = 7936
NST = 126
MAIN = NST * CBLK
VPR = 128 // PAD
GRP = CBLK // VPR
T3_ROWS = VOCAB // VPR


def _mlp_body(tt_hbm, mc_ref, bias_ref, out_ref, buf, sem):
    i = pl.program_id(0)

    @pl.when(i == 0)
    def _():
        pltpu.make_async_copy(tt_hbm.at[:, pl.ds(0, CBLK)], buf.at[0],
                              sem.at[0]).start()
        pltpu.make_async_copy(tt_hbm.at[:, pl.ds(CBLK, CBLK)], buf.at[1],
                              sem.at[1]).start()

    @pl.when(i + 2 <= NST - 1)
    def _():
        ns = lax.rem(i + 2, 3)
        pltpu.make_async_copy(tt_hbm.at[:, pl.ds((i + 2) * CBLK, CBLK)],
                              buf.at[ns], sem.at[ns]).start()

    slot = lax.rem(i, 3)
    pltpu.make_async_copy(tt_hbm.at[:, pl.ds(i * CBLK, CBLK)], buf.at[slot],
                          sem.at[slot]).wait()
    mc = mc_ref[...]
    bias = bias_ref[...]
    for m in range(VPR):
        tbm = buf[slot, :, m * GRP:(m + 1) * GRP]
        o = lax.dot_general(tbm, mc, (((0,), (1,)), ((), ())),
                            preferred_element_type=jnp.float32)
        o = o + bias
        out_ref[:, m * PAD:(m + 1) * PAD] = jnp.where(o > 0.0, o,
                                                      jnp.exp(o) - 1.0)


def _transform_table(tt, Mc, bias):
    return pl.pallas_call(
        _mlp_body,
        grid=(NST,),
        in_specs=[
            pl.BlockSpec(memory_space=pl.ANY),
            pl.BlockSpec((PAD, EMB), lambda i: (0, 0)),
            pl.BlockSpec((1, PAD), lambda i: (0, 0)),
        ],
        out_specs=pl.BlockSpec((GRP, 128), lambda i: (i, 0)),
        out_shape=jax.ShapeDtypeStruct((T3_ROWS, 128), jnp.float32),
        scratch_shapes=[
            pltpu.VMEM((3, EMB, CBLK), jnp.float32),
            pltpu.SemaphoreType.DMA((3,)),
        ],
        compiler_params=pltpu.CompilerParams(
            dimension_semantics=("arbitrary",),
        ),
    )(tt, Mc, bias)


def _tail_body(main_ref, tail_ref, out_ref):
    out_ref[...] = tail_ref[...]


def _patch_tail(t3main, tail8):
    return pl.pallas_call(
        _tail_body,
        grid=(1,),
        in_specs=[
            pl.BlockSpec(memory_space=pl.ANY),
            pl.BlockSpec((8, 128), lambda i: (0, 0)),
        ],
        out_specs=pl.BlockSpec((8, 128), lambda i: (T3_ROWS // 8 - 1, 0)),
        out_shape=jax.ShapeDtypeStruct((T3_ROWS, 128), jnp.float32),
        input_output_aliases={0: 0},
    )(t3main, tail8)


NC, NS = 2, 16
NW = NC * NS
PER_W = N_TOK // NW
CH = 3200
N_CH = PER_W // CH


def _gather_body(table_hbm, idx_hbm, out_hbm, idx_v, rows_v, sem):
    wid = lax.axis_index("s") * NC + lax.axis_index("c")
    base = wid * PER_W
    for j in range(N_CH):
        off = base + j * CH
        pltpu.sync_copy(idx_hbm.at[pl.ds(off, CH)], idx_v)
        pltpu.async_copy(table_hbm.at[idx_v], rows_v, sem).wait()
        pltpu.sync_copy(rows_v, out_hbm.at[pl.ds(off, CH)])


@functools.cache
def _make_gather():
    return pl.kernel(
        _gather_body,
        mesh=plsc.VectorSubcoreMesh(core_axis_name="c", subcore_axis_name="s"),
        out_type=jax.ShapeDtypeStruct((N_TOK, PAD), jnp.float32),
        scratch_types=[
            pltpu.VMEM((CH,), jnp.int32),
            pltpu.VMEM((CH, PAD), jnp.float32),
            pltpu.SemaphoreType.DMA,
        ],
        compiler_params=pltpu.CompilerParams(use_tc_tiling_on_sc=False),
    )


def kernel(src, table, W1, b1, W2, b2):
    W2p = jnp.zeros((PAD, EMB), jnp.float32).at[:OUT].set(W2)
    b2p = jnp.zeros((PAD,), jnp.float32).at[:OUT].set(b2)
    Mc = jnp.dot(W2p, W1, precision=lax.Precision.HIGHEST)
    bias = (W2p @ b1 + b2p).reshape(1, PAD)

    t3main = _transform_table(table.T, Mc, bias)

    ttail = lax.slice(table, (MAIN, 0), (VOCAB, EMB))
    to = jnp.dot(ttail, Mc.T, precision=lax.Precision.HIGHEST) + bias
    tail8 = jnp.where(to > 0.0, to, jnp.expm1(to)).reshape(8, 128)
    t3 = _patch_tail(t3main, tail8)

    v = src.reshape(N_TOK)
    r = v % CBLK
    idx2 = jnp.where(v < MAIN,
                     ((v // CBLK) * GRP + r % GRP) * VPR + r // GRP,
                     v).astype(jnp.int32)

    rows = _make_gather()(t3.reshape(VOCAB, PAD), idx2)
    return rows[:, :OUT].reshape(B, L, OUT)


# speedup vs baseline: 1.3227x; 1.0031x over previous
import functools

import jax
import jax.numpy as jnp
from jax import lax
from jax.experimental import pallas as pl
from jax.experimental.pallas import tpu as pltpu
from jax.experimental.pallas import tpu_sc as plsc

VOCAB = 1000000
EMB = 64
OUT = 6
PAD = 16
B, L = 4096, 200
N_TOK = B * L

CBLK = 7936
NST = 126
MAIN = NST * CBLK
VPR = 128 // PAD
GRP = CBLK // VPR
T3_ROWS = VOCAB // VPR


def _mlp_body(tt_hbm, mc_ref, bias_ref, out_ref, buf, sem):
    i = pl.program_id(0)

    @pl.when(i == 0)
    def _():
        pltpu.make_async_copy(tt_hbm.at[:, pl.ds(0, CBLK)], buf.at[0],
                              sem.at[0]).start()
        pltpu.make_async_copy(tt_hbm.at[:, pl.ds(CBLK, CBLK)], buf.at[1],
                              sem.at[1]).start()

    @pl.when(i + 2 <= NST - 1)
    def _():
        ns = lax.rem(i + 2, 3)
        pltpu.make_async_copy(tt_hbm.at[:, pl.ds((i + 2) * CBLK, CBLK)],
                              buf.at[ns], sem.at[ns]).start()

    slot = lax.rem(i, 3)
    pltpu.make_async_copy(tt_hbm.at[:, pl.ds(i * CBLK, CBLK)], buf.at[slot],
                          sem.at[slot]).wait()
    mc = mc_ref[...]
    bias = bias_ref[...]
    for m in range(VPR):
        tbm = buf[slot, :, m * GRP:(m + 1) * GRP]
        o = lax.dot_general(tbm, mc, (((0,), (0,)), ((), ())),
                            preferred_element_type=jnp.float32)
        o = o + bias
        out_ref[:, m * PAD:(m + 1) * PAD] = jnp.where(o > 0.0, o,
                                                      jnp.exp(o) - 1.0)


def _transform_table(tt, Mc, bias):
    return pl.pallas_call(
        _mlp_body,
        grid=(NST,),
        in_specs=[
            pl.BlockSpec(memory_space=pl.ANY),
            pl.BlockSpec((EMB, PAD), lambda i: (0, 0)),
            pl.BlockSpec((1, PAD), lambda i: (0, 0)),
        ],
        out_specs=pl.BlockSpec((GRP, 128), lambda i: (i, 0)),
        out_shape=jax.ShapeDtypeStruct((T3_ROWS, 128), jnp.float32),
        scratch_shapes=[
            pltpu.VMEM((3, EMB, CBLK), jnp.float32),
            pltpu.SemaphoreType.DMA((3,)),
        ],
        compiler_params=pltpu.CompilerParams(
            dimension_semantics=("arbitrary",),
        ),
    )(tt, Mc, bias)


def _tail_body(main_ref, tail_ref, out_ref):
    out_ref[...] = tail_ref[...]


def _patch_tail(t3main, tail8):
    return pl.pallas_call(
        _tail_body,
        grid=(1,),
        in_specs=[
            pl.BlockSpec(memory_space=pl.ANY),
            pl.BlockSpec((8, 128), lambda i: (0, 0)),
        ],
        out_specs=pl.BlockSpec((8, 128), lambda i: (T3_ROWS // 8 - 1, 0)),
        out_shape=jax.ShapeDtypeStruct((T3_ROWS, 128), jnp.float32),
        input_output_aliases={0: 0},
    )(t3main, tail8)


NC, NS = 2, 16
NW = NC * NS
PER_W = N_TOK // NW
CH = 3200
N_CH = PER_W // CH


def _gather_body(table_hbm, idx_hbm, out_hbm, idx_v, rows_v, sem):
    wid = lax.axis_index("s") * NC + lax.axis_index("c")
    base = wid * PER_W
    for j in range(N_CH):
        off = base + j * CH
        pltpu.sync_copy(idx_hbm.at[pl.ds(off, CH)], idx_v)
        pltpu.async_copy(table_hbm.at[idx_v], rows_v, sem).wait()
        pltpu.sync_copy(rows_v, out_hbm.at[pl.ds(off, CH)])


@functools.cache
def _make_gather():
    return pl.kernel(
        _gather_body,
        mesh=plsc.VectorSubcoreMesh(core_axis_name="c", subcore_axis_name="s"),
        out_type=jax.ShapeDtypeStruct((N_TOK, PAD), jnp.float32),
        scratch_types=[
            pltpu.VMEM((CH,), jnp.int32),
            pltpu.VMEM((CH, PAD), jnp.float32),
            pltpu.SemaphoreType.DMA,
        ],
        compiler_params=pltpu.CompilerParams(use_tc_tiling_on_sc=False),
    )


def kernel(src, table, W1, b1, W2, b2):
    W2p = jnp.zeros((PAD, EMB), jnp.float32).at[:OUT].set(W2)
    b2p = jnp.zeros((PAD,), jnp.float32).at[:OUT].set(b2)
    Mc = jnp.dot(W2p, W1, precision=lax.Precision.HIGHEST)
    bias = (W2p @ b1 + b2p).reshape(1, PAD)

    t3main = _transform_table(table.T, Mc.T, bias)

    ttail = lax.slice(table, (MAIN, 0), (VOCAB, EMB))
    to = jnp.dot(ttail, Mc.T, precision=lax.Precision.HIGHEST) + bias
    tail8 = jnp.where(to > 0.0, to, jnp.expm1(to)).reshape(8, 128)
    t3 = _patch_tail(t3main, tail8)

    v = src.reshape(N_TOK)
    r = v % CBLK
    idx2 = jnp.where(v < MAIN,
                     ((v // CBLK) * GRP + r % GRP) * VPR + r // GRP,
                     v).astype(jnp.int32)

    rows = _make_gather()(t3.reshape(VOCAB, PAD), idx2)
    return rows[:, :OUT].reshape(B, L, OUT)


# speedup vs baseline: 1.5389x; 1.1634x over previous
import functools

import jax
import jax.numpy as jnp
from jax import lax
from jax.experimental import pallas as pl
from jax.experimental.pallas import tpu as pltpu
from jax.experimental.pallas import tpu_sc as plsc

VOCAB = 1000000
EMB = 64
OUT = 6
PAD = 16
B, L = 4096, 200
N_TOK = B * L

CBLK = 15872
NST = 63
MAIN = NST * CBLK
VPR = 128 // PAD
GRP = CBLK // VPR
T3_ROWS = VOCAB // VPR


def _mlp_body(tt_hbm, mc_ref, bias_ref, out_ref, buf, sem):
    i = pl.program_id(0)

    @pl.when(i == 0)
    def _():
        pltpu.make_async_copy(tt_hbm.at[:, pl.ds(0, CBLK)], buf.at[0],
                              sem.at[0]).start()
        pltpu.make_async_copy(tt_hbm.at[:, pl.ds(CBLK, CBLK)], buf.at[1],
                              sem.at[1]).start()

    @pl.when(i + 2 <= NST - 1)
    def _():
        ns = lax.rem(i + 2, 3)
        pltpu.make_async_copy(tt_hbm.at[:, pl.ds((i + 2) * CBLK, CBLK)],
                              buf.at[ns], sem.at[ns]).start()

    slot = lax.rem(i, 3)
    pltpu.make_async_copy(tt_hbm.at[:, pl.ds(i * CBLK, CBLK)], buf.at[slot],
                          sem.at[slot]).wait()
    mc = mc_ref[...]
    bias = bias_ref[...]
    for m in range(VPR):
        tbm = buf[slot, :, m * GRP:(m + 1) * GRP]
        o = lax.dot_general(tbm, mc, (((0,), (0,)), ((), ())),
                            preferred_element_type=jnp.float32)
        o = o + bias
        out_ref[:, m * PAD:(m + 1) * PAD] = jnp.where(o > 0.0, o,
                                                      jnp.exp(o) - 1.0)


def _transform_table(tt, Mc, bias):
    return pl.pallas_call(
        _mlp_body,
        grid=(NST,),
        in_specs=[
            pl.BlockSpec(memory_space=pl.ANY),
            pl.BlockSpec((EMB, PAD), lambda i: (0, 0)),
            pl.BlockSpec((1, PAD), lambda i: (0, 0)),
        ],
        out_specs=pl.BlockSpec((GRP, 128), lambda i: (i, 0)),
        out_shape=jax.ShapeDtypeStruct((T3_ROWS, 128), jnp.float32),
        scratch_shapes=[
            pltpu.VMEM((3, EMB, CBLK), jnp.float32),
            pltpu.SemaphoreType.DMA((3,)),
        ],
        compiler_params=pltpu.CompilerParams(
            dimension_semantics=("arbitrary",),
        ),
    )(tt, Mc, bias)


def _tail_body(main_ref, tail_ref, out_ref):
    out_ref[...] = tail_ref[...]


def _patch_tail(t3main, tail8):
    return pl.pallas_call(
        _tail_body,
        grid=(1,),
        in_specs=[
            pl.BlockSpec(memory_space=pl.ANY),
            pl.BlockSpec((8, 128), lambda i: (0, 0)),
        ],
        out_specs=pl.BlockSpec((8, 128), lambda i: (T3_ROWS // 8 - 1, 0)),
        out_shape=jax.ShapeDtypeStruct((T3_ROWS, 128), jnp.float32),
        input_output_aliases={0: 0},
    )(t3main, tail8)


NC, NS = 2, 16
NW = NC * NS
PER_W = N_TOK // NW
CH = 3200
N_CH = PER_W // CH


def _gather_body(table_hbm, idx_hbm, out_hbm, idx_v, rows_v, sem):
    wid = lax.axis_index("s") * NC + lax.axis_index("c")
    base = wid * PER_W
    for j in range(N_CH):
        off = base + j * CH
        pltpu.sync_copy(idx_hbm.at[pl.ds(off, CH)], idx_v)
        pltpu.async_copy(table_hbm.at[idx_v], rows_v, sem).wait()
        pltpu.sync_copy(rows_v, out_hbm.at[pl.ds(off, CH)])


@functools.cache
def _make_gather():
    return pl.kernel(
        _gather_body,
        mesh=plsc.VectorSubcoreMesh(core_axis_name="c", subcore_axis_name="s"),
        out_type=jax.ShapeDtypeStruct((N_TOK, PAD), jnp.float32),
        scratch_types=[
            pltpu.VMEM((CH,), jnp.int32),
            pltpu.VMEM((CH, PAD), jnp.float32),
            pltpu.SemaphoreType.DMA,
        ],
        compiler_params=pltpu.CompilerParams(use_tc_tiling_on_sc=False),
    )


def kernel(src, table, W1, b1, W2, b2):
    W2p = jnp.zeros((PAD, EMB), jnp.float32).at[:OUT].set(W2)
    b2p = jnp.zeros((PAD,), jnp.float32).at[:OUT].set(b2)
    Mc = jnp.dot(W2p, W1, precision=lax.Precision.HIGHEST)
    bias = (W2p @ b1 + b2p).reshape(1, PAD)

    t3main = _transform_table(table.T, Mc.T, bias)

    ttail = lax.slice(table, (MAIN, 0), (VOCAB, EMB))
    to = jnp.dot(ttail, Mc.T, precision=lax.Precision.HIGHEST) + bias
    tail8 = jnp.where(to > 0.0, to, jnp.expm1(to)).reshape(8, 128)
    t3 = _patch_tail(t3main, tail8)

    v = src.reshape(N_TOK)
    r = v % CBLK
    idx2 = jnp.where(v < MAIN,
                     ((v // CBLK) * GRP + r % GRP) * VPR + r // GRP,
                     v).astype(jnp.int32)

    rows = _make_gather()(t3.reshape(VOCAB, PAD), idx2)
    return rows[:, :OUT].reshape(B, L, OUT)
